# Initial kernel scaffold; baseline (speedup 1.0000x reference)
import jax
import jax.numpy as jnp
from jax.experimental import pallas as pl


def kernel(data, depth, edge_index_d6, edge_type_d6, edge_index_d5, edge_type_d5, edge_index_d4, edge_type_d4, edge_index_d3, edge_type_d3, edge_index_d2, edge_type_d2, parent_d6, parent_d5, parent_d4, parent_d3, params):
    raise NotImplementedError("write your pallas kernel here")


# speedup vs baseline: 2.0111x; 2.0111x over previous
import functools

import jax
import jax.numpy as jnp
from jax import lax
from jax.experimental import pallas as pl
from jax.experimental.pallas import tpu as pltpu
from jax.experimental.pallas import tpu_sc as plsc

NLVL = {6: 100000, 5: 12500, 4: 1563, 3: 196, 2: 25}

_LANES = 128
_EBLK = 512
_NSUB = 16
_NCORE = 2


def _rup(x, m):
    return (x + m - 1) // m * m


def _mesh():
    return plsc.VectorSubcoreMesh(
        core_axis_name="c", subcore_axis_name="s",
        num_cores=_NCORE, num_subcores=_NSUB)


@functools.lru_cache(maxsize=None)
def _sc_edge_scatter(n_nodes, c2, e_pad):
    n_pad = _rup(n_nodes + 1, _LANES)
    bpt = e_pad // (_NSUB * _EBLK)
    wq = n_pad // _NSUB

    def body(ytab, gidx2, dst3, zrows, out, gv, dv, rows, acc, sem):
        c = lax.axis_index("c")
        s = lax.axis_index("s")
        pltpu.sync_copy(zrows.at[pl.ds(s * wq, wq)], acc.at[pl.ds(s * wq, wq)])
        plsc.subcore_barrier()

        def blk(i, carry):
            r0 = (s * bpt + i) * 4
            pltpu.sync_copy(gidx2.at[pl.ds(c, 1), pl.ds(r0, 4), :], gv)
            pltpu.sync_copy(dst3.at[pl.ds(0, 1), pl.ds(r0, 4), :], dv)
            cps = [pltpu.async_copy(ytab.at[gv.at[0, j]],
                                    rows.at[pl.ds(j * _LANES, _LANES)], sem)
                   for j in range(4)]
            for cp in cps:
                cp.wait()
            for j in range(4):
                pltpu.sync_copy(rows.at[pl.ds(j * _LANES, _LANES)],
                                acc.at[dv.at[0, j]], add=True)
            return carry

        lax.fori_loop(0, bpt, blk, 0)
        plsc.subcore_barrier()
        pltpu.sync_copy(acc.at[pl.ds(s * wq, wq)],
                        out.at[pl.ds(c * n_pad + s * wq, wq)])

    return pl.kernel(
        body,
        out_type=jax.ShapeDtypeStruct((2 * n_pad, c2), jnp.float32),
        mesh=_mesh(),
        compiler_params=pltpu.CompilerParams(use_tc_tiling_on_sc=False),
        scratch_types=[
            pltpu.VMEM((1, 4, _LANES), jnp.int32),
            pltpu.VMEM((1, 4, _LANES), jnp.int32),
            pltpu.VMEM((_EBLK, c2), jnp.float32),
            pltpu.VMEM_SHARED((n_pad, c2), jnp.float32),
            pltpu.SemaphoreType.DMA,
        ])


@functools.lru_cache(maxsize=None)
def _sc_down(nc_pad, n_par, cch):
    np_pad = _rup(n_par + 1, _LANES)
    bpt = nc_pad // (_NCORE * _NSUB * _EBLK)
    wq = np_pad // _NSUB

    def body(xp, par3, zs, zc, ones, s_out, c_out, pv, rows, onesv, accs, accc, sem):
        c = lax.axis_index("c")
        s = lax.axis_index("s")
        pltpu.sync_copy(zs.at[pl.ds(s * wq, wq)], accs.at[pl.ds(s * wq, wq)])
        pltpu.sync_copy(zc.at[pl.ds(s * wq, wq)], accc.at[pl.ds(s * wq, wq)])
        pltpu.sync_copy(ones, onesv)
        plsc.subcore_barrier()
        w = c * _NSUB + s

        def blk(i, carry):
            b0 = (w * bpt + i)
            pltpu.sync_copy(xp.at[pl.ds(b0 * _EBLK, _EBLK)], rows)
            pltpu.sync_copy(par3.at[pl.ds(0, 1), pl.ds(b0 * 4, 4), :], pv)
            for j in range(4):
                pltpu.sync_copy(rows.at[pl.ds(j * _LANES, _LANES)],
                                accs.at[pv.at[0, j]], add=True)
                pltpu.sync_copy(onesv, accc.at[pv.at[0, j]], add=True)
            return carry

        lax.fori_loop(0, bpt, blk, 0)
        plsc.subcore_barrier()
        pltpu.sync_copy(accs.at[pl.ds(s * wq, wq)],
                        s_out.at[pl.ds(c * np_pad + s * wq, wq)])
        pltpu.sync_copy(accc.at[pl.ds(s * wq, wq)],
                        c_out.at[pl.ds(c * np_pad + s * wq, wq)])

    return pl.kernel(
        body,
        out_type=[jax.ShapeDtypeStruct((2 * np_pad, cch), jnp.float32),
                  jax.ShapeDtypeStruct((2 * np_pad, 16), jnp.float32)],
        mesh=_mesh(),
        compiler_params=pltpu.CompilerParams(use_tc_tiling_on_sc=False),
        scratch_types=[
            pltpu.VMEM((1, 4, _LANES), jnp.int32),
            pltpu.VMEM((_EBLK, cch), jnp.float32),
            pltpu.VMEM((_LANES, 16), jnp.float32),
            pltpu.VMEM_SHARED((np_pad, cch), jnp.float32),
            pltpu.VMEM_SHARED((np_pad, 16), jnp.float32),
            pltpu.SemaphoreType.DMA,
        ])


@functools.lru_cache(maxsize=None)
def _sc_up(nc_pad, n_par, cch):
    bpw = nc_pad // (_NCORE * _NSUB * _LANES)

    def body(xtab, par3, out, pv, rows, sem):
        c = lax.axis_index("c")
        s = lax.axis_index("s")
        w = c * _NSUB + s

        def blk(i, carry):
            r = w * bpw + i
            pltpu.sync_copy(par3.at[pl.ds(0, 1), pl.ds(r, 1), :], pv)
            pltpu.async_copy(xtab.at[pv.at[0, 0]], rows, sem).wait()
            pltpu.sync_copy(rows, out.at[pl.ds(r * _LANES, _LANES)])
            return carry

        lax.fori_loop(0, bpw, blk, 0)

    return pl.kernel(
        body,
        out_type=jax.ShapeDtypeStruct((nc_pad, cch), jnp.float32),
        mesh=_mesh(),
        compiler_params=pltpu.CompilerParams(use_tc_tiling_on_sc=False),
        scratch_types=[
            pltpu.VMEM((1, 1, _LANES), jnp.int32),
            pltpu.VMEM((_LANES, cch), jnp.float32),
            pltpu.SemaphoreType.DMA,
        ])


def _mm_typed(x, w):
    n, cin = x.shape
    t, _, cout = w.shape
    bn = 512

    def body(x_ref, w_ref, o_ref):
        o_ref[0] = jnp.dot(x_ref[...], w_ref[0],
                           preferred_element_type=jnp.float32)

    return pl.pallas_call(
        body,
        grid=(t, pl.cdiv(n, bn)),
        in_specs=[pl.BlockSpec((bn, cin), lambda tt, i: (i, 0)),
                  pl.BlockSpec((1, cin, cout), lambda tt, i: (tt, 0, 0))],
        out_specs=pl.BlockSpec((1, bn, cout), lambda tt, i: (tt, i, 0)),
        out_shape=jax.ShapeDtypeStruct((t, n, cout), jnp.float32),
    )(x, w)


def _gn_gelu(h, gamma, beta, res):
    cout = h.shape[1]
    groups = max(1, min(32, cout // 4))
    gsz = cout // groups
    ci = lax.broadcasted_iota(jnp.int32, (cout, groups), 0)
    gi = lax.broadcasted_iota(jnp.int32, (cout, groups), 1)
    pool = (ci // gsz == gi).astype(jnp.float32)
    inv = 1.0 / gsz
    mu = jnp.dot(h, pool, preferred_element_type=jnp.float32) * inv
    mub = jnp.dot(mu, pool.T, preferred_element_type=jnp.float32)
    d = h - mub
    var = jnp.dot(d * d, pool, preferred_element_type=jnp.float32) * inv
    varb = jnp.dot(var, pool.T, preferred_element_type=jnp.float32)
    y = d * lax.rsqrt(varb + 1e-5) * gamma + beta
    if res is not None:
        y = y + res
    return jax.nn.gelu(y)


def _epilogue(mode, a, w, b, gamma, beta, agg0=None, agg1=None,
              cnt0=None, cnt1=None, res=None, skip=None):
    n = a.shape[0] if a is not None else agg0.shape[0]
    cin, cout = w.shape
    bn = 512
    nb = pl.cdiv(n, bn)

    ins = [w.reshape(1, cin, cout), b.reshape(1, cout),
           gamma.reshape(1, cout), beta.reshape(1, cout)]
    specs = [pl.BlockSpec((1, cin, cout), lambda i: (0, 0, 0)),
             pl.BlockSpec((1, cout), lambda i: (0, 0)),
             pl.BlockSpec((1, cout), lambda i: (0, 0)),
             pl.BlockSpec((1, cout), lambda i: (0, 0))]

    def row_spec(c):
        return pl.BlockSpec((bn, c), lambda i: (i, 0))

    if a is not None:
        ins.append(a)
        specs.append(row_spec(cin))
    if agg0 is not None:
        ins += [agg0, agg1]
        specs += [row_spec(agg0.shape[1]), row_spec(agg1.shape[1])]
    if cnt0 is not None:
        ins += [cnt0, cnt1]
        specs += [row_spec(16), row_spec(16)]
    if res is not None:
        ins.append(res)
        specs.append(row_spec(cout))
    if skip is not None:
        ins.append(skip)
        specs.append(row_spec(cout))

    def body(*refs):
        it = iter(refs)
        w_r, b_r, g_r, bt_r = next(it), next(it), next(it), next(it)
        a_r = next(it) if a is not None else None
        ag0_r = next(it) if agg0 is not None else None
        ag1_r = next(it) if agg0 is not None else None
        c0_r = next(it) if cnt0 is not None else None
        c1_r = next(it) if cnt0 is not None else None
        res_r = next(it) if res is not None else None
        skip_r = next(it) if skip is not None else None
        o_ref = next(it)

        if mode == "down":
            ssum = ag0_r[...] + ag1_r[...]
            cnt = c0_r[...][:, 0:1] + c1_r[...][:, 0:1]
            ain = ssum / jnp.maximum(cnt, 1.0)
        else:
            ain = a_r[...]
        h = jnp.dot(ain, w_r[0], preferred_element_type=jnp.float32) + b_r[0]
        if mode in ("conv", "res"):
            h = h + jnp.concatenate([ag0_r[...], ag1_r[...]], axis=1)
        out = _gn_gelu(h, g_r[0], bt_r[0],
                       res_r[...] if res_r is not None else None)
        if skip_r is not None:
            out = out + skip_r[...]
        o_ref[...] = out

    return pl.pallas_call(
        body,
        grid=(nb,),
        in_specs=specs,
        out_specs=row_spec(cout),
        out_shape=jax.ShapeDtypeStruct((n, cout), jnp.float32),
    )(*ins)


def _prep_edges(ei, et, n):
    e = et.shape[0]
    e_pad = _rup(e, _NSUB * _EBLK)
    pad = e_pad - e
    src = jnp.concatenate([ei[0], jnp.zeros((pad,), jnp.int32)])
    dst = jnp.concatenate([ei[1], jnp.full((pad,), n, jnp.int32)])
    etp = jnp.concatenate([et, jnp.zeros((pad,), jnp.int32)])
    g = (etp * n + src) * 2
    gidx2 = jnp.stack([g, g + 1]).reshape(2, e_pad // _LANES, _LANES)
    dst3 = dst.reshape(1, e_pad // _LANES, _LANES)
    return gidx2, dst3, e_pad


def _graph_conv_agg(x, p, edges, n):
    gidx2, dst3, e_pad = edges
    cout = p["W"].shape[2]
    c2 = cout // 2
    y = _mm_typed(x, p["W"])
    ytab = y.reshape(7 * n * 2, c2)
    n_pad = _rup(n + 1, _LANES)
    zrows = jnp.zeros((n_pad, c2), jnp.float32)
    acc = _sc_edge_scatter(n, c2, e_pad)(ytab, gidx2, dst3, zrows)
    acc = acc.reshape(2, n_pad, c2)
    return acc[0, :n], acc[1, :n]


def _conv_na(x, p, edges, n, res=None):
    agg0, agg1 = _graph_conv_agg(x, p, edges, n)
    mode = "res" if res is not None else "conv"
    return _epilogue(mode, x, p["Wself"], p["b"], p["g"], p["beta"],
                     agg0=agg0, agg1=agg1, res=res)


def _resblk(x, p, edges, n):
    h = _conv_na(x, p["c1"], edges, n)
    return _conv_na(h, p["c2"], edges, n, res=x)


def _down(x, p, par, n_child, n_par):
    cch = x.shape[1]
    nc_pad = _rup(n_child, _NCORE * _NSUB * _EBLK)
    xp = jnp.pad(x, ((0, nc_pad - n_child), (0, 0)))
    parp = jnp.concatenate(
        [par, jnp.full((nc_pad - n_child,), n_par, jnp.int32)])
    par3 = parp.reshape(1, nc_pad // _LANES, _LANES)
    np_pad = _rup(n_par + 1, _LANES)
    zs = jnp.zeros((np_pad, cch), jnp.float32)
    zc = jnp.zeros((np_pad, 16), jnp.float32)
    ones = jnp.zeros((_LANES, 16), jnp.float32).at[:, 0].set(1.0)
    s2, c2 = _sc_down(nc_pad, n_par, cch)(xp, par3, zs, zc, ones)
    s2 = s2.reshape(2, np_pad, cch)
    c2 = c2.reshape(2, np_pad, 16)
    return _epilogue("down", None, p["W"], p["b"], p["g"], p["beta"],
                     agg0=s2[0, :n_par], agg1=s2[1, :n_par],
                     cnt0=c2[0, :n_par], cnt1=c2[1, :n_par])


def _up(x, p, par, n_child, skip):
    cch = x.shape[1]
    nc_pad = _rup(n_child, _NCORE * _NSUB * _LANES)
    parp = jnp.concatenate([par, jnp.zeros((nc_pad - n_child,), jnp.int32)])
    par3 = parp.reshape(1, nc_pad // _LANES, _LANES)
    gath = _sc_up(nc_pad, x.shape[0], cch)(x, par3)
    return _epilogue("up", gath[:n_child], p["W"], p["b"], p["g"], p["beta"],
                     skip=skip)


def kernel(data, depth, edge_index_d6, edge_type_d6, edge_index_d5,
           edge_type_d5, edge_index_d4, edge_type_d4, edge_index_d3,
           edge_type_d3, edge_index_d2, edge_type_d2, parent_d6, parent_d5,
           parent_d4, parent_d3, params):
    ei = {6: edge_index_d6, 5: edge_index_d5, 4: edge_index_d4,
          3: edge_index_d3, 2: edge_index_d2}
    et = {6: edge_type_d6, 5: edge_type_d5, 4: edge_type_d4,
          3: edge_type_d3, 2: edge_type_d2}
    par = {6: parent_d6, 5: parent_d5, 4: parent_d4, 3: parent_d3}

    edges = {d: _prep_edges(ei[d], et[d], NLVL[d]) for d in (6, 5, 4, 3, 2)}

    x = _conv_na(data, params["conv1"], edges[6], NLVL[6])
    d = 6
    for st in params["enc"]:
        x = _conv_na(x, st["conv"], edges[d], NLVL[d])
        x = _down(x, st["down"], par[d], NLVL[d], NLVL[d - 1])
        d -= 1

    convs = {4: x}
    for i, dd in enumerate([4, 3, 2]):
        h = convs[dd]
        for rp in params["net"]["enc"][i]:
            h = _resblk(h, rp, edges[dd], NLVL[dd])
        convs[dd] = h
        if i < 2:
            convs[dd - 1] = _down(h, params["net"]["down"][i], par[dd],
                                  NLVL[dd], NLVL[dd - 1])
    out = convs[2]
    for i, dd in enumerate([2, 3, 4]):
        for rp in params["net"]["dec"][i]:
            out = _resblk(out, rp, edges[dd], NLVL[dd])
        if i < 2:
            out = _up(out, params["net"]["up"][i], par[dd + 1],
                      NLVL[dd + 1], convs[dd + 1])
    return out
